# baseline (speedup 1.0000x reference)
import functools

import jax
import jax.numpy as jnp
from jax import lax
from jax.experimental import pallas as pl
from jax.experimental.pallas import tpu as pltpu
from jax.experimental.pallas import tpu_sc as plsc

DUSER = 100000
DITEM = 1000000
DEMB = 64
DHIDDEN = 256
BATCH = 4096

_NC = 2
_NS = 16
_NW = _NC * _NS
_BPW = BATCH // _NW
_L = 16


def _sc_gather_body(u2, i2, uid, iid, u_out, i_out,
                    idxu, idxi, tidu, tidi, ru, ri, semu, semi):
    wid = lax.axis_index("s") * _NC + lax.axis_index("c")
    base = wid * _BPW
    pltpu.sync_copy(uid.at[pl.ds(base, _BPW)], idxu)
    pltpu.sync_copy(iid.at[pl.ds(base, _BPW)], idxi)
    for c in range(_BPW // _L):
        sl = pl.ds(c * _L, _L)
        tidu[sl] = idxu[sl] >> 1
        tidi[sl] = idxi[sl] >> 1
    cu = pltpu.async_copy(u2.at[tidu], ru, semu)
    ci = pltpu.async_copy(i2.at[tidi], ri, semi)
    cu.wait()
    ci.wait()
    pltpu.sync_copy(ru, u_out.at[pl.ds(base, _BPW)])
    pltpu.sync_copy(ri, i_out.at[pl.ds(base, _BPW)])


@functools.cache
def _sc_gather():
    return pl.kernel(
        _sc_gather_body,
        out_type=[
            jax.ShapeDtypeStruct((BATCH, 2 * DEMB), jnp.float32),
            jax.ShapeDtypeStruct((BATCH, 2 * DEMB), jnp.float32),
        ],
        mesh=plsc.VectorSubcoreMesh(
            core_axis_name="c", subcore_axis_name="s",
            num_cores=_NC, num_subcores=_NS),
        scratch_types=[
            pltpu.VMEM((_BPW,), jnp.int32),
            pltpu.VMEM((_BPW,), jnp.int32),
            pltpu.VMEM((_BPW,), jnp.int32),
            pltpu.VMEM((_BPW,), jnp.int32),
            pltpu.VMEM((_BPW, 2 * DEMB), jnp.float32),
            pltpu.VMEM((_BPW, 2 * DEMB), jnp.float32),
            pltpu.SemaphoreType.DMA,
            pltpu.SemaphoreType.DMA,
        ],
    )


def _mlp_body(u_ref, i_ref, mu_ref, mi_ref, w1u_ref, w1i_ref, b1_ref,
              w2_ref, b2_ref, w3_ref, b3_ref, out_ref):
    mu = mu_ref[...]
    mi = mi_ref[...]
    u = mu * u_ref[:, :DEMB] + (1.0 - mu) * u_ref[:, DEMB:]
    i = mi * i_ref[:, :DEMB] + (1.0 - mi) * i_ref[:, DEMB:]
    h = (jnp.dot(u, w1u_ref[...], preferred_element_type=jnp.float32)
         + jnp.dot(i, w1i_ref[...], preferred_element_type=jnp.float32)
         + b1_ref[...])
    h = jax.nn.sigmoid(h)
    h = jax.nn.sigmoid(
        jnp.dot(h, w2_ref[...], preferred_element_type=jnp.float32)
        + b2_ref[...])
    out_ref[...] = jax.nn.sigmoid(
        jnp.dot(h, w3_ref[...], preferred_element_type=jnp.float32)
        + b3_ref[...])


def _mlp(u_blk, i_blk, mu, mi, w1u, w1i, b1, w2, b2, w3, b3, block_b=512):
    grid = (BATCH // block_b,)
    full = lambda *s: pl.BlockSpec(s, lambda j: (0,) * len(s))
    return pl.pallas_call(
        _mlp_body,
        grid=grid,
        in_specs=[
            pl.BlockSpec((block_b, 2 * DEMB), lambda j: (j, 0)),
            pl.BlockSpec((block_b, 2 * DEMB), lambda j: (j, 0)),
            pl.BlockSpec((block_b, 1), lambda j: (j, 0)),
            pl.BlockSpec((block_b, 1), lambda j: (j, 0)),
            full(DEMB, DHIDDEN),
            full(DEMB, DHIDDEN),
            full(1, DHIDDEN),
            full(DHIDDEN, DHIDDEN),
            full(1, DHIDDEN),
            full(DHIDDEN, 1),
            full(1, 1),
        ],
        out_specs=pl.BlockSpec((block_b, 1), lambda j: (j, 0)),
        out_shape=jax.ShapeDtypeStruct((BATCH, 1), jnp.float32),
    )(u_blk, i_blk, mu, mi, w1u, w1i, b1, w2, b2, w3, b3)


def kernel(user_id, item_id, user_table, item_table, W1, b1, W2, b2, W3, b3):
    uid = user_id.astype(jnp.int32)
    iid = item_id.astype(jnp.int32)
    u2 = user_table.reshape(DUSER // 2, 2 * DEMB)
    i2 = item_table.reshape(DITEM // 2, 2 * DEMB)
    u_blk, i_blk = _sc_gather()(u2, i2, uid, iid)
    mu = (1 - (uid % 2)).astype(jnp.float32).reshape(BATCH, 1)
    mi = (1 - (iid % 2)).astype(jnp.float32).reshape(BATCH, 1)
    return _mlp(u_blk, i_blk, mu, mi,
                W1[:DEMB], W1[DEMB:],
                b1.reshape(1, DHIDDEN), W2, b2.reshape(1, DHIDDEN),
                W3, b3.reshape(1, 1))


# speedup vs baseline: 1.1094x; 1.1094x over previous
import functools

import jax
import jax.numpy as jnp
from jax import lax
from jax.experimental import pallas as pl
from jax.experimental.pallas import tpu as pltpu
from jax.experimental.pallas import tpu_sc as plsc

DUSER = 100000
DITEM = 1000000
DEMB = 64
DHIDDEN = 256
BATCH = 4096

_NC = 2
_NS = 16
_NW = _NC * _NS
_BPW = BATCH // _NW


def _sc_gather_body(u2, i2, uid, iid, u_out, i_out,
                    idxu, idxi, ru, ri, semu, semi):
    wid = lax.axis_index("s") * _NC + lax.axis_index("c")
    base = wid * _BPW
    pltpu.sync_copy(uid.at[pl.ds(base, _BPW)], idxu)
    pltpu.sync_copy(iid.at[pl.ds(base, _BPW)], idxi)
    cu = pltpu.async_copy(u2.at[idxu], ru, semu)
    ci = pltpu.async_copy(i2.at[idxi], ri, semi)
    cu.wait()
    ci.wait()
    pltpu.sync_copy(ru, u_out.at[pl.ds(base, _BPW)])
    pltpu.sync_copy(ri, i_out.at[pl.ds(base, _BPW)])


@functools.cache
def _sc_gather():
    return pl.kernel(
        _sc_gather_body,
        out_type=[
            jax.ShapeDtypeStruct((BATCH, 2 * DEMB), jnp.float32),
            jax.ShapeDtypeStruct((BATCH, 2 * DEMB), jnp.float32),
        ],
        mesh=plsc.VectorSubcoreMesh(
            core_axis_name="c", subcore_axis_name="s",
            num_cores=_NC, num_subcores=_NS),
        scratch_types=[
            pltpu.VMEM((_BPW,), jnp.int32),
            pltpu.VMEM((_BPW,), jnp.int32),
            pltpu.VMEM((_BPW, 2 * DEMB), jnp.float32),
            pltpu.VMEM((_BPW, 2 * DEMB), jnp.float32),
            pltpu.SemaphoreType.DMA,
            pltpu.SemaphoreType.DMA,
        ],
    )


def _mlp_body(u_ref, i_ref, w1u_ref, w1i_ref, b1_ref,
              w2_ref, b2_ref, w3_ref, b3_ref, out_ref):
    u = u_ref[:, :DEMB]
    i = i_ref[:, :DEMB]
    h = (jnp.dot(u, w1u_ref[...], preferred_element_type=jnp.float32)
         + jnp.dot(i, w1i_ref[...], preferred_element_type=jnp.float32)
         + b1_ref[...])
    h = jax.nn.sigmoid(h)
    h = jax.nn.sigmoid(
        jnp.dot(h, w2_ref[...], preferred_element_type=jnp.float32)
        + b2_ref[...])
    out_ref[...] = jax.nn.sigmoid(
        jnp.dot(h, w3_ref[...], preferred_element_type=jnp.float32)
        + b3_ref[...])


def _mlp(u_blk, i_blk, w1u, w1i, b1, w2, b2, w3, b3, block_b=512):
    grid = (BATCH // block_b,)
    full = lambda *s: pl.BlockSpec(s, lambda j: (0,) * len(s))
    return pl.pallas_call(
        _mlp_body,
        grid=grid,
        in_specs=[
            pl.BlockSpec((block_b, 2 * DEMB), lambda j: (j, 0)),
            pl.BlockSpec((block_b, 2 * DEMB), lambda j: (j, 0)),
            full(DEMB, DHIDDEN),
            full(DEMB, DHIDDEN),
            full(1, DHIDDEN),
            full(DHIDDEN, DHIDDEN),
            full(1, DHIDDEN),
            full(DHIDDEN, 1),
            full(1, 1),
        ],
        out_specs=pl.BlockSpec((block_b, 1), lambda j: (j, 0)),
        out_shape=jax.ShapeDtypeStruct((BATCH, 1), jnp.float32),
    )(u_blk, i_blk, w1u, w1i, b1, w2, b2, w3, b3)


def kernel(user_id, item_id, user_table, item_table, W1, b1, W2, b2, W3, b3):
    uid = user_id.astype(jnp.int32)
    iid = item_id.astype(jnp.int32)
    u2 = jnp.pad(user_table, ((0, 0), (0, DEMB)))
    i2 = jnp.pad(item_table, ((0, 0), (0, DEMB)))
    u_blk, i_blk = _sc_gather()(u2, i2, uid, iid)
    return _mlp(u_blk, i_blk,
                W1[:DEMB], W1[DEMB:],
                b1.reshape(1, DHIDDEN), W2, b2.reshape(1, DHIDDEN),
                W3, b3.reshape(1, 1))


# speedup vs baseline: 1.6599x; 1.4962x over previous
import functools

import jax
import jax.numpy as jnp
from jax import lax
from jax.experimental import pallas as pl
from jax.experimental.pallas import tpu as pltpu
from jax.experimental.pallas import tpu_sc as plsc

DUSER = 100000
DITEM = 1000000
DEMB = 64
DHIDDEN = 256
BATCH = 4096

_NC = 2
_NS = 16
_NW = _NC * _NS
_BPW = BATCH // _NW


def _row_dmas(table, idx_v, dst, sem):
    lane = lax.iota(jnp.int32, 16)
    copies = []
    for c in range(_BPW // 16):
        chunk = idx_v[pl.ds(c * 16, 16)]
        for j in range(16):
            rid = jnp.sum(jnp.where(lane == j, chunk, 0))
            i = c * 16 + j
            copies.append(pltpu.async_copy(
                table.at[pl.ds(rid, 1)], dst.at[pl.ds(i, 1)], sem))
    return copies


def _sc_gather_body(u2, i2, uid, iid, u_out, i_out,
                    idxu, idxi, ru, ri, semu, semi):
    wid = lax.axis_index("s") * _NC + lax.axis_index("c")
    base = wid * _BPW
    pltpu.sync_copy(uid.at[pl.ds(base, _BPW)], idxu)
    pltpu.sync_copy(iid.at[pl.ds(base, _BPW)], idxi)
    copies = _row_dmas(u2, idxu, ru, semu) + _row_dmas(i2, idxi, ri, semi)
    for cp in copies:
        cp.wait()
    pltpu.sync_copy(ru, u_out.at[pl.ds(base, _BPW)])
    pltpu.sync_copy(ri, i_out.at[pl.ds(base, _BPW)])


@functools.cache
def _sc_gather():
    return pl.kernel(
        _sc_gather_body,
        out_type=[
            jax.ShapeDtypeStruct((BATCH, DEMB), jnp.float32),
            jax.ShapeDtypeStruct((BATCH, DEMB), jnp.float32),
        ],
        mesh=plsc.VectorSubcoreMesh(
            core_axis_name="c", subcore_axis_name="s",
            num_cores=_NC, num_subcores=_NS),
        compiler_params=pltpu.CompilerParams(needs_layout_passes=False),
        scratch_types=[
            pltpu.VMEM((_BPW,), jnp.int32),
            pltpu.VMEM((_BPW,), jnp.int32),
            pltpu.VMEM((_BPW, DEMB), jnp.float32),
            pltpu.VMEM((_BPW, DEMB), jnp.float32),
            pltpu.SemaphoreType.DMA,
            pltpu.SemaphoreType.DMA,
        ],
    )


def _mlp_body(u_ref, i_ref, w1u_ref, w1i_ref, b1_ref,
              w2_ref, b2_ref, w3_ref, b3_ref, out_ref):
    u = u_ref[...]
    i = i_ref[...]
    h = (jnp.dot(u, w1u_ref[...], preferred_element_type=jnp.float32)
         + jnp.dot(i, w1i_ref[...], preferred_element_type=jnp.float32)
         + b1_ref[...])
    h = jax.nn.sigmoid(h)
    h = jax.nn.sigmoid(
        jnp.dot(h, w2_ref[...], preferred_element_type=jnp.float32)
        + b2_ref[...])
    out_ref[...] = jax.nn.sigmoid(
        jnp.dot(h, w3_ref[...], preferred_element_type=jnp.float32)
        + b3_ref[...])


def _mlp(u_blk, i_blk, w1u, w1i, b1, w2, b2, w3, b3, block_b=512):
    grid = (BATCH // block_b,)
    full = lambda *s: pl.BlockSpec(s, lambda j: (0,) * len(s))
    return pl.pallas_call(
        _mlp_body,
        grid=grid,
        in_specs=[
            pl.BlockSpec((block_b, DEMB), lambda j: (j, 0)),
            pl.BlockSpec((block_b, DEMB), lambda j: (j, 0)),
            full(DEMB, DHIDDEN),
            full(DEMB, DHIDDEN),
            full(1, DHIDDEN),
            full(DHIDDEN, DHIDDEN),
            full(1, DHIDDEN),
            full(DHIDDEN, 1),
            full(1, 1),
        ],
        out_specs=pl.BlockSpec((block_b, 1), lambda j: (j, 0)),
        out_shape=jax.ShapeDtypeStruct((BATCH, 1), jnp.float32),
    )(u_blk, i_blk, w1u, w1i, b1, w2, b2, w3, b3)


def kernel(user_id, item_id, user_table, item_table, W1, b1, W2, b2, W3, b3):
    uid = user_id.astype(jnp.int32)
    iid = item_id.astype(jnp.int32)
    u_blk, i_blk = _sc_gather()(user_table, item_table, uid, iid)
    return _mlp(u_blk, i_blk,
                W1[:DEMB], W1[DEMB:],
                b1.reshape(1, DHIDDEN), W2, b2.reshape(1, DHIDDEN),
                W3, b3.reshape(1, 1))
